# Initial kernel scaffold; baseline (speedup 1.0000x reference)
import jax
import jax.numpy as jnp
from jax.experimental import pallas as pl


def kernel(fatoms, fbonds, agraph, bgraph, segment_ids, mol_sizes, W_i, W_h, W_o, b_o):
    raise NotImplementedError("write your pallas kernel here")


# speedup vs baseline: 3.1836x; 3.1836x over previous
import functools

import jax
import jax.numpy as jnp
from jax import lax
from jax.experimental import pallas as pl
from jax.experimental.pallas import tpu as pltpu
from jax.experimental.pallas import tpu_sc as plsc

E = 320000
N = 10000
NB = 6
H = 128
AF = 136
BI = 148
DEPTH = 3
NMOL = 500
APM = 20

CH = 64
IDX_ROWS = CH * NB // 128
NW = 32


def _gather_sum(n_items):
    n_chunks = (n_items + CH - 1) // CH
    n_rounds = (n_chunks + NW - 1) // NW
    out_rows = n_chunks * CH
    mesh = plsc.VectorSubcoreMesh(core_axis_name="c", subcore_axis_name="s")

    @functools.partial(
        pl.kernel,
        mesh=mesh,
        out_type=jax.ShapeDtypeStruct((out_rows, H), jnp.float32),
        scratch_types=[
            pltpu.VMEM((IDX_ROWS, 128), jnp.int32),
            pltpu.VMEM((CH * NB, H), jnp.float32),
            pltpu.VMEM((CH, H), jnp.float32),
            pltpu.SemaphoreType.DMA,
        ],
    )
    def k(table_hbm, idx_hbm, out_hbm, idx_v, rows_v, out_v, sem):
        wid = lax.axis_index("s") * 2 + lax.axis_index("c")

        def round_body(t, carry):
            cid = t * NW + wid

            @pl.when(cid < n_chunks)
            def _():
                pltpu.sync_copy(idx_hbm.at[cid], idx_v)
                cps = [
                    pltpu.async_copy(table_hbm.at[idx_v.at[b]],
                                     rows_v.at[pl.ds(b * 128, 128)], sem)
                    for b in range(IDX_ROWS)
                ]
                for c in cps:
                    c.wait()

                def edge_body(e, carry2):
                    r = e * NB
                    for h in range(H // 16):
                        acc = rows_v[r, pl.ds(h * 16, 16)]
                        for kk in range(1, NB):
                            acc = acc + rows_v[r + kk, pl.ds(h * 16, 16)]
                        out_v[e, pl.ds(h * 16, 16)] = acc
                    return carry2

                lax.fori_loop(0, CH, edge_body, 0)
                pltpu.sync_copy(out_v, out_hbm.at[pl.ds(cid * CH, CH)])

            return carry

        lax.fori_loop(0, n_rounds, round_body, 0)

    return k


def _mm_in_body(fb_ref, wi_ref, bin_ref, msg_ref):
    b = lax.dot_general(fb_ref[...], wi_ref[...], (((1,), (1,)), ((), ())),
                        preferred_element_type=jnp.float32)
    bin_ref[...] = b
    msg_ref[...] = jnp.maximum(b, 0.0)


def _mm_step_body(s_ref, bin_ref, wh_ref, msg_ref):
    m = lax.dot_general(s_ref[...], wh_ref[...], (((1,), (1,)), ((), ())),
                        preferred_element_type=jnp.float32)
    msg_ref[...] = jnp.maximum(bin_ref[...] + m, 0.0)


def _mm_out_body(fa_ref, a_ref, woa_ref, woh_ref, bo_ref, ms_ref, out_ref):
    h = lax.dot_general(fa_ref[...], woa_ref[...], (((1,), (1,)), ((), ())),
                        preferred_element_type=jnp.float32)
    h += lax.dot_general(a_ref[...], woh_ref[...], (((1,), (1,)), ((), ())),
                         preferred_element_type=jnp.float32)
    h = jnp.maximum(h + bo_ref[...], 0.0)
    hs = jnp.sum(h.reshape(h.shape[0] // APM, APM, H), axis=1)
    out_ref[...] = hs / ms_ref[...]


BE = 2000


def kernel(fatoms, fbonds, agraph, bgraph, segment_ids, mol_sizes,
           W_i, W_h, W_o, b_o):
    binput, msg = pl.pallas_call(
        _mm_in_body,
        grid=(E // BE,),
        in_specs=[
            pl.BlockSpec((BE, BI), lambda i: (i, 0)),
            pl.BlockSpec((H, BI), lambda i: (0, 0)),
        ],
        out_specs=[
            pl.BlockSpec((BE, H), lambda i: (i, 0)),
            pl.BlockSpec((BE, H), lambda i: (i, 0)),
        ],
        out_shape=[
            jax.ShapeDtypeStruct((E, H), jnp.float32),
            jax.ShapeDtypeStruct((E, H), jnp.float32),
        ],
    )(fbonds, W_i)

    bidx3d = bgraph.reshape(E // CH, IDX_ROWS, 128)
    n_apad = 10240
    aflat = jnp.pad(agraph.reshape(-1), (0, (n_apad - N) * NB))
    aidx3d = aflat.reshape(n_apad // CH, IDX_ROWS, 128)

    gs_bonds = _gather_sum(E)
    gs_atoms = _gather_sum(n_apad)

    for _ in range(DEPTH - 1):
        s = gs_bonds(msg, bidx3d)
        msg = pl.pallas_call(
            _mm_step_body,
            grid=(E // BE,),
            in_specs=[
                pl.BlockSpec((BE, H), lambda i: (i, 0)),
                pl.BlockSpec((BE, H), lambda i: (i, 0)),
                pl.BlockSpec((H, H), lambda i: (0, 0)),
            ],
            out_specs=pl.BlockSpec((BE, H), lambda i: (i, 0)),
            out_shape=jax.ShapeDtypeStruct((E, H), jnp.float32),
        )(s, binput, W_h)

    a = gs_atoms(msg, aidx3d)[:N]

    woa = W_o[:, :AF]
    woh = W_o[:, AF:]
    bo2d = b_o.reshape(1, H)
    msb = jnp.broadcast_to(mol_sizes.reshape(NMOL, 1), (NMOL, H))
    out = pl.pallas_call(
        _mm_out_body,
        grid=(1,),
        in_specs=[
            pl.BlockSpec((N, AF), lambda i: (0, 0)),
            pl.BlockSpec((N, H), lambda i: (0, 0)),
            pl.BlockSpec((H, AF), lambda i: (0, 0)),
            pl.BlockSpec((H, H), lambda i: (0, 0)),
            pl.BlockSpec((1, H), lambda i: (0, 0)),
            pl.BlockSpec((NMOL, H), lambda i: (0, 0)),
        ],
        out_specs=pl.BlockSpec((NMOL, H), lambda i: (0, 0)),
        out_shape=jax.ShapeDtypeStruct((NMOL, H), jnp.float32),
    )(fatoms, a, woa, woh, bo2d, msb)
    return out


# speedup vs baseline: 4.3213x; 1.3574x over previous
import functools

import jax
import jax.numpy as jnp
from jax import lax
from jax.experimental import pallas as pl
from jax.experimental.pallas import tpu as pltpu
from jax.experimental.pallas import tpu_sc as plsc

E = 320000
N = 10000
NB = 6
H = 128
AF = 136
BI = 148
DEPTH = 3
NMOL = 500
APM = 20

CH = 64
IDX_ROWS = CH * NB // 128
NW = 32


def _gather_sum(n_items):
    n_chunks = (n_items + CH - 1) // CH
    n_rounds = (n_chunks + NW - 1) // NW
    n_pairs = (n_rounds + 1) // 2
    out_rows = n_chunks * CH
    mesh = plsc.VectorSubcoreMesh(core_axis_name="c", subcore_axis_name="s")

    @functools.partial(
        pl.kernel,
        mesh=mesh,
        out_type=jax.ShapeDtypeStruct((out_rows, H), jnp.float32),
        scratch_types=[
            pltpu.VMEM((2, IDX_ROWS, 128), jnp.int32),
            pltpu.VMEM((2, CH * NB, H), jnp.float32),
            pltpu.VMEM((CH, H), jnp.float32),
            pltpu.SemaphoreType.DMA,
            pltpu.SemaphoreType.DMA,
            pltpu.SemaphoreType.DMA,
            pltpu.SemaphoreType.DMA,
        ],
    )
    def k(table_hbm, idx_hbm, out_hbm, idx_v, rows_v, out_v,
          isem0, isem1, rsem0, rsem1):
        wid = lax.axis_index("s") * 2 + lax.axis_index("c")
        isems = [isem0, isem1]
        rsems = [rsem0, rsem1]

        def valid(t):
            return (t * NW + wid) < n_chunks

        def fire_idx(t, b):
            @pl.when(valid(t))
            def _():
                pltpu.async_copy(idx_hbm.at[t * NW + wid], idx_v.at[b],
                                 isems[b])

        def wait_idx(t, b):
            @pl.when(valid(t))
            def _():
                pltpu.make_async_copy(idx_hbm.at[0], idx_v.at[b],
                                      isems[b]).wait()

        def fire_rows(t, b):
            @pl.when(valid(t))
            def _():
                for r in range(IDX_ROWS):
                    pltpu.async_copy(
                        table_hbm.at[idx_v.at[b].at[r]],
                        rows_v.at[b].at[pl.ds(r * 128, 128)], rsems[b])

        def wait_rows(t, b):
            @pl.when(valid(t))
            def _():
                pltpu.make_async_copy(table_hbm.at[pl.ds(0, CH * NB)],
                                      rows_v.at[b], rsems[b]).wait()

        def compute(t, b):
            @pl.when(valid(t))
            def _():
                rows = rows_v.at[b]

                def edge_body(i, carry2):
                    for u in range(2):
                        e = i * 2 + u
                        r = e * NB
                        for h in range(H // 16):
                            acc = rows[r, pl.ds(h * 16, 16)]
                            for kk in range(1, NB):
                                acc = acc + rows[r + kk, pl.ds(h * 16, 16)]
                            out_v[e, pl.ds(h * 16, 16)] = acc
                    return carry2

                lax.fori_loop(0, CH // 2, edge_body, 0)
                pltpu.sync_copy(out_v,
                                out_hbm.at[pl.ds((t * NW + wid) * CH, CH)])

        fire_idx(0, 0)
        wait_idx(0, 0)
        fire_rows(0, 0)
        fire_idx(1, 1)

        def pair_body(i, carry):
            for bb in range(2):
                t = i * 2 + bb
                p = bb
                q = 1 - bb
                wait_idx(t + 1, q)
                fire_rows(t + 1, q)
                wait_rows(t, p)
                fire_idx(t + 2, p)
                compute(t, p)
            return carry

        lax.fori_loop(0, n_pairs, pair_body, 0)

    return k


def _mm_in_body(fb_ref, wi_ref, bin_ref, msg_ref):
    b = lax.dot_general(fb_ref[...], wi_ref[...], (((1,), (1,)), ((), ())),
                        preferred_element_type=jnp.float32)
    bin_ref[...] = b
    msg_ref[...] = jnp.maximum(b, 0.0)


def _mm_step_body(s_ref, bin_ref, wh_ref, msg_ref):
    m = lax.dot_general(s_ref[...], wh_ref[...], (((1,), (1,)), ((), ())),
                        preferred_element_type=jnp.float32)
    msg_ref[...] = jnp.maximum(bin_ref[...] + m, 0.0)


def _mm_out_body(fa_ref, a_ref, woa_ref, woh_ref, bo_ref, ms_ref, out_ref):
    h = lax.dot_general(fa_ref[...], woa_ref[...], (((1,), (1,)), ((), ())),
                        preferred_element_type=jnp.float32)
    h += lax.dot_general(a_ref[...], woh_ref[...], (((1,), (1,)), ((), ())),
                         preferred_element_type=jnp.float32)
    h = jnp.maximum(h + bo_ref[...], 0.0)
    hs = jnp.sum(h.reshape(h.shape[0] // APM, APM, H), axis=1)
    out_ref[...] = hs / ms_ref[...]


BE = 2000


def kernel(fatoms, fbonds, agraph, bgraph, segment_ids, mol_sizes,
           W_i, W_h, W_o, b_o):
    binput, msg = pl.pallas_call(
        _mm_in_body,
        grid=(E // BE,),
        in_specs=[
            pl.BlockSpec((BE, BI), lambda i: (i, 0)),
            pl.BlockSpec((H, BI), lambda i: (0, 0)),
        ],
        out_specs=[
            pl.BlockSpec((BE, H), lambda i: (i, 0)),
            pl.BlockSpec((BE, H), lambda i: (i, 0)),
        ],
        out_shape=[
            jax.ShapeDtypeStruct((E, H), jnp.float32),
            jax.ShapeDtypeStruct((E, H), jnp.float32),
        ],
    )(fbonds, W_i)

    bidx3d = bgraph.reshape(E // CH, IDX_ROWS, 128)
    n_apad = 10240
    aflat = jnp.pad(agraph.reshape(-1), (0, (n_apad - N) * NB))
    aidx3d = aflat.reshape(n_apad // CH, IDX_ROWS, 128)

    gs_bonds = _gather_sum(E)
    gs_atoms = _gather_sum(n_apad)

    for _ in range(DEPTH - 1):
        s = gs_bonds(msg, bidx3d)
        msg = pl.pallas_call(
            _mm_step_body,
            grid=(E // BE,),
            in_specs=[
                pl.BlockSpec((BE, H), lambda i: (i, 0)),
                pl.BlockSpec((BE, H), lambda i: (i, 0)),
                pl.BlockSpec((H, H), lambda i: (0, 0)),
            ],
            out_specs=pl.BlockSpec((BE, H), lambda i: (i, 0)),
            out_shape=jax.ShapeDtypeStruct((E, H), jnp.float32),
        )(s, binput, W_h)

    a = gs_atoms(msg, aidx3d)[:N]

    woa = W_o[:, :AF]
    woh = W_o[:, AF:]
    bo2d = b_o.reshape(1, H)
    msb = jnp.broadcast_to(mol_sizes.reshape(NMOL, 1), (NMOL, H))
    out = pl.pallas_call(
        _mm_out_body,
        grid=(1,),
        in_specs=[
            pl.BlockSpec((N, AF), lambda i: (0, 0)),
            pl.BlockSpec((N, H), lambda i: (0, 0)),
            pl.BlockSpec((H, AF), lambda i: (0, 0)),
            pl.BlockSpec((H, H), lambda i: (0, 0)),
            pl.BlockSpec((1, H), lambda i: (0, 0)),
            pl.BlockSpec((NMOL, H), lambda i: (0, 0)),
        ],
        out_specs=pl.BlockSpec((NMOL, H), lambda i: (0, 0)),
        out_shape=jax.ShapeDtypeStruct((NMOL, H), jnp.float32),
    )(fatoms, a, woa, woh, bo2d, msb)
    return out


# speedup vs baseline: 4.8345x; 1.1188x over previous
import functools

import jax
import jax.numpy as jnp
from jax import lax
from jax.experimental import pallas as pl
from jax.experimental.pallas import tpu as pltpu
from jax.experimental.pallas import tpu_sc as plsc

E = 320000
N = 10000
NB = 6
H = 128
AF = 136
BI = 148
DEPTH = 3
NMOL = 500
APM = 20

CH = 64
IDX_ROWS = CH * NB // 128
NW = 32


def _gather_sum(n_items):
    n_chunks = (n_items + CH - 1) // CH
    n_rounds = (n_chunks + NW - 1) // NW
    n_pairs = (n_rounds + 1) // 2
    out_rows = n_chunks * CH
    mesh = plsc.VectorSubcoreMesh(core_axis_name="c", subcore_axis_name="s")

    @functools.partial(
        pl.kernel,
        mesh=mesh,
        out_type=jax.ShapeDtypeStruct((out_rows, H), jnp.float32),
        scratch_types=[
            pltpu.VMEM((2, IDX_ROWS, 128), jnp.int32),
            pltpu.VMEM((2, CH * NB, H), jnp.float32),
            pltpu.VMEM((2, CH, H), jnp.float32),
            pltpu.SemaphoreType.DMA,
            pltpu.SemaphoreType.DMA,
            pltpu.SemaphoreType.DMA,
            pltpu.SemaphoreType.DMA,
            pltpu.SemaphoreType.DMA,
            pltpu.SemaphoreType.DMA,
        ],
    )
    def k(table_hbm, idx_hbm, out_hbm, idx_v, rows_v, out_v,
          isem0, isem1, rsem0, rsem1, osem0, osem1):
        wid = lax.axis_index("s") * 2 + lax.axis_index("c")
        isems = [isem0, isem1]
        rsems = [rsem0, rsem1]
        osems = [osem0, osem1]

        def valid(t):
            return (t * NW + wid) < n_chunks

        def fire_idx(t, b):
            @pl.when(valid(t))
            def _():
                pltpu.async_copy(idx_hbm.at[t * NW + wid], idx_v.at[b],
                                 isems[b])

        def wait_idx(t, b):
            @pl.when(valid(t))
            def _():
                pltpu.make_async_copy(idx_hbm.at[0], idx_v.at[b],
                                      isems[b]).wait()

        def fire_rows(t, b):
            @pl.when(valid(t))
            def _():
                for r in range(IDX_ROWS):
                    pltpu.async_copy(
                        table_hbm.at[idx_v.at[b].at[r]],
                        rows_v.at[b].at[pl.ds(r * 128, 128)], rsems[b])

        def wait_rows(t, b):
            @pl.when(valid(t))
            def _():
                pltpu.make_async_copy(table_hbm.at[pl.ds(0, CH * NB)],
                                      rows_v.at[b], rsems[b]).wait()

        def wait_out(t, b):
            @pl.when(jnp.logical_and(t >= 0, valid(t)))
            def _():
                pltpu.make_async_copy(out_v.at[b],
                                      out_hbm.at[pl.ds(0, CH)],
                                      osems[b]).wait()

        def compute(t, b):
            @pl.when(valid(t))
            def _():
                rows = rows_v.at[b]
                outb = out_v.at[b]

                def edge_body(i, carry2):
                    for u in range(2):
                        e = i * 2 + u
                        r = e * NB
                        for h in range(H // 16):
                            sl = pl.ds(h * 16, 16)
                            t0 = rows[r, sl] + rows[r + 1, sl]
                            t1 = rows[r + 2, sl] + rows[r + 3, sl]
                            t2 = rows[r + 4, sl] + rows[r + 5, sl]
                            outb[e, sl] = (t0 + t1) + t2
                    return carry2

                lax.fori_loop(0, CH // 2, edge_body, 0)
                pltpu.async_copy(outb,
                                 out_hbm.at[pl.ds((t * NW + wid) * CH, CH)],
                                 osems[b])

        fire_idx(0, 0)
        wait_idx(0, 0)
        fire_rows(0, 0)
        fire_idx(1, 1)

        def pair_body(i, carry):
            for bb in range(2):
                t = i * 2 + bb
                p = bb
                q = 1 - bb
                wait_idx(t + 1, q)
                fire_rows(t + 1, q)
                wait_rows(t, p)
                fire_idx(t + 2, p)
                wait_out(t - 2, p)
                compute(t, p)
            return carry

        lax.fori_loop(0, n_pairs, pair_body, 0)
        wait_out(n_pairs * 2 - 2, 0)
        wait_out(n_pairs * 2 - 1, 1)

    return k


def _mm_in_body(fb_ref, wi_ref, bin_ref, msg_ref):
    b = lax.dot_general(fb_ref[...], wi_ref[...], (((1,), (1,)), ((), ())),
                        preferred_element_type=jnp.float32)
    bin_ref[...] = b
    msg_ref[...] = jnp.maximum(b, 0.0)


def _mm_step_body(s_ref, bin_ref, wh_ref, msg_ref):
    m = lax.dot_general(s_ref[...], wh_ref[...], (((1,), (1,)), ((), ())),
                        preferred_element_type=jnp.float32)
    msg_ref[...] = jnp.maximum(bin_ref[...] + m, 0.0)


def _mm_out_body(fa_ref, a_ref, woa_ref, woh_ref, bo_ref, ms_ref, out_ref):
    h = lax.dot_general(fa_ref[...], woa_ref[...], (((1,), (1,)), ((), ())),
                        preferred_element_type=jnp.float32)
    h += lax.dot_general(a_ref[...], woh_ref[...], (((1,), (1,)), ((), ())),
                         preferred_element_type=jnp.float32)
    h = jnp.maximum(h + bo_ref[...], 0.0)
    hs = jnp.sum(h.reshape(h.shape[0] // APM, APM, H), axis=1)
    out_ref[...] = hs / ms_ref[...]


BE = 2000


def kernel(fatoms, fbonds, agraph, bgraph, segment_ids, mol_sizes,
           W_i, W_h, W_o, b_o):
    binput, msg = pl.pallas_call(
        _mm_in_body,
        grid=(E // BE,),
        in_specs=[
            pl.BlockSpec((BE, BI), lambda i: (i, 0)),
            pl.BlockSpec((H, BI), lambda i: (0, 0)),
        ],
        out_specs=[
            pl.BlockSpec((BE, H), lambda i: (i, 0)),
            pl.BlockSpec((BE, H), lambda i: (i, 0)),
        ],
        out_shape=[
            jax.ShapeDtypeStruct((E, H), jnp.float32),
            jax.ShapeDtypeStruct((E, H), jnp.float32),
        ],
    )(fbonds, W_i)

    bidx3d = bgraph.reshape(E // CH, IDX_ROWS, 128)
    n_apad = 10240
    aflat = jnp.pad(agraph.reshape(-1), (0, (n_apad - N) * NB))
    aidx3d = aflat.reshape(n_apad // CH, IDX_ROWS, 128)

    gs_bonds = _gather_sum(E)
    gs_atoms = _gather_sum(n_apad)

    for _ in range(DEPTH - 1):
        s = gs_bonds(msg, bidx3d)
        msg = pl.pallas_call(
            _mm_step_body,
            grid=(E // BE,),
            in_specs=[
                pl.BlockSpec((BE, H), lambda i: (i, 0)),
                pl.BlockSpec((BE, H), lambda i: (i, 0)),
                pl.BlockSpec((H, H), lambda i: (0, 0)),
            ],
            out_specs=pl.BlockSpec((BE, H), lambda i: (i, 0)),
            out_shape=jax.ShapeDtypeStruct((E, H), jnp.float32),
        )(s, binput, W_h)

    a = gs_atoms(msg, aidx3d)[:N]

    woa = W_o[:, :AF]
    woh = W_o[:, AF:]
    bo2d = b_o.reshape(1, H)
    msb = jnp.broadcast_to(mol_sizes.reshape(NMOL, 1), (NMOL, H))
    out = pl.pallas_call(
        _mm_out_body,
        grid=(1,),
        in_specs=[
            pl.BlockSpec((N, AF), lambda i: (0, 0)),
            pl.BlockSpec((N, H), lambda i: (0, 0)),
            pl.BlockSpec((H, AF), lambda i: (0, 0)),
            pl.BlockSpec((H, H), lambda i: (0, 0)),
            pl.BlockSpec((1, H), lambda i: (0, 0)),
            pl.BlockSpec((NMOL, H), lambda i: (0, 0)),
        ],
        out_specs=pl.BlockSpec((NMOL, H), lambda i: (0, 0)),
        out_shape=jax.ShapeDtypeStruct((NMOL, H), jnp.float32),
    )(fatoms, a, woa, woh, bo2d, msb)
    return out


# speedup vs baseline: 6.3767x; 1.3190x over previous
import functools

import jax
import jax.numpy as jnp
from jax import lax
from jax.experimental import pallas as pl
from jax.experimental.pallas import tpu as pltpu
from jax.experimental.pallas import tpu_sc as plsc

E = 320000
N = 10000
NB = 6
H = 128
AF = 136
BI = 148
DEPTH = 3
NMOL = 500
APM = 20

CH = 64
IDX_ROWS = CH * NB // 128
NW = 32


def _gather_sum(n_items):
    n_chunks = (n_items + CH - 1) // CH
    n_rounds = (n_chunks + NW - 1) // NW
    n_pairs = (n_rounds + 1) // 2
    out_rows = n_chunks * CH
    mesh = plsc.VectorSubcoreMesh(core_axis_name="c", subcore_axis_name="s")

    @functools.partial(
        pl.kernel,
        mesh=mesh,
        out_type=jax.ShapeDtypeStruct((out_rows, H), jnp.float32),
        scratch_types=[
            pltpu.VMEM((2, IDX_ROWS, 128), jnp.int32),
            pltpu.VMEM((2, CH * NB, H), jnp.float32),
            pltpu.VMEM((2, CH, H), jnp.float32),
            pltpu.SemaphoreType.DMA,
            pltpu.SemaphoreType.DMA,
            pltpu.SemaphoreType.DMA,
            pltpu.SemaphoreType.DMA,
            pltpu.SemaphoreType.DMA,
            pltpu.SemaphoreType.DMA,
        ],
    )
    def k(table_hbm, idx_hbm, out_hbm, idx_v, rows_v, out_v,
          isem0, isem1, rsem0, rsem1, osem0, osem1):
        wid = lax.axis_index("s") * 2 + lax.axis_index("c")
        isems = [isem0, isem1]
        rsems = [rsem0, rsem1]
        osems = [osem0, osem1]

        def valid(t):
            return (t * NW + wid) < n_chunks

        def fire_idx(t, b):
            @pl.when(valid(t))
            def _():
                pltpu.async_copy(idx_hbm.at[t * NW + wid], idx_v.at[b],
                                 isems[b])

        def wait_idx(t, b):
            @pl.when(valid(t))
            def _():
                pltpu.make_async_copy(idx_hbm.at[0], idx_v.at[b],
                                      isems[b]).wait()

        def fire_rows(t, b):
            @pl.when(valid(t))
            def _():
                for r in range(IDX_ROWS):
                    pltpu.async_copy(
                        table_hbm.at[idx_v.at[b].at[r]],
                        rows_v.at[b].at[pl.ds(r * 128, 128)], rsems[b])

        def wait_rows(t, b):
            @pl.when(valid(t))
            def _():
                pltpu.make_async_copy(table_hbm.at[pl.ds(0, CH * NB)],
                                      rows_v.at[b], rsems[b]).wait()

        def wait_out(t, b):
            @pl.when(jnp.logical_and(t >= 0, valid(t)))
            def _():
                pltpu.make_async_copy(out_v.at[b],
                                      out_hbm.at[pl.ds(0, CH)],
                                      osems[b]).wait()

        def compute(t, b):
            @pl.when(valid(t))
            def _():
                rows = rows_v.at[b]
                outb = out_v.at[b]

                @plsc.parallel_loop(0, CH, 1, unroll=2)
                def _edge(e):
                    r = e * NB
                    for h in range(H // 16):
                        sl = pl.ds(h * 16, 16)
                        t0 = rows[r, sl] + rows[r + 1, sl]
                        t1 = rows[r + 2, sl] + rows[r + 3, sl]
                        t2 = rows[r + 4, sl] + rows[r + 5, sl]
                        outb[e, sl] = (t0 + t1) + t2
                pltpu.async_copy(outb,
                                 out_hbm.at[pl.ds((t * NW + wid) * CH, CH)],
                                 osems[b])

        fire_idx(0, 0)
        wait_idx(0, 0)
        fire_rows(0, 0)
        fire_idx(1, 1)

        def pair_body(i, carry):
            for bb in range(2):
                t = i * 2 + bb
                p = bb
                q = 1 - bb
                wait_idx(t + 1, q)
                fire_rows(t + 1, q)
                wait_rows(t, p)
                fire_idx(t + 2, p)
                wait_out(t - 2, p)
                compute(t, p)
            return carry

        lax.fori_loop(0, n_pairs, pair_body, 0)
        wait_out(n_pairs * 2 - 2, 0)
        wait_out(n_pairs * 2 - 1, 1)

    return k


def _mm_in_body(fb_ref, wi_ref, bin_ref, msg_ref):
    b = lax.dot_general(fb_ref[...], wi_ref[...], (((1,), (1,)), ((), ())),
                        preferred_element_type=jnp.float32)
    bin_ref[...] = b
    msg_ref[...] = jnp.maximum(b, 0.0)


def _mm_step_body(s_ref, bin_ref, wh_ref, msg_ref):
    m = lax.dot_general(s_ref[...], wh_ref[...], (((1,), (1,)), ((), ())),
                        preferred_element_type=jnp.float32)
    msg_ref[...] = jnp.maximum(bin_ref[...] + m, 0.0)


def _mm_out_body(fa_ref, a_ref, woa_ref, woh_ref, bo_ref, ms_ref, out_ref):
    h = lax.dot_general(fa_ref[...], woa_ref[...], (((1,), (1,)), ((), ())),
                        preferred_element_type=jnp.float32)
    h += lax.dot_general(a_ref[...], woh_ref[...], (((1,), (1,)), ((), ())),
                         preferred_element_type=jnp.float32)
    h = jnp.maximum(h + bo_ref[...], 0.0)
    hs = jnp.sum(h.reshape(h.shape[0] // APM, APM, H), axis=1)
    out_ref[...] = hs / ms_ref[...]


BE = 2000


def kernel(fatoms, fbonds, agraph, bgraph, segment_ids, mol_sizes,
           W_i, W_h, W_o, b_o):
    binput, msg = pl.pallas_call(
        _mm_in_body,
        grid=(E // BE,),
        in_specs=[
            pl.BlockSpec((BE, BI), lambda i: (i, 0)),
            pl.BlockSpec((H, BI), lambda i: (0, 0)),
        ],
        out_specs=[
            pl.BlockSpec((BE, H), lambda i: (i, 0)),
            pl.BlockSpec((BE, H), lambda i: (i, 0)),
        ],
        out_shape=[
            jax.ShapeDtypeStruct((E, H), jnp.float32),
            jax.ShapeDtypeStruct((E, H), jnp.float32),
        ],
    )(fbonds, W_i)

    bidx3d = bgraph.reshape(E // CH, IDX_ROWS, 128)
    n_apad = 10240
    aflat = jnp.pad(agraph.reshape(-1), (0, (n_apad - N) * NB))
    aidx3d = aflat.reshape(n_apad // CH, IDX_ROWS, 128)

    gs_bonds = _gather_sum(E)
    gs_atoms = _gather_sum(n_apad)

    for _ in range(DEPTH - 1):
        s = gs_bonds(msg, bidx3d)
        msg = pl.pallas_call(
            _mm_step_body,
            grid=(E // BE,),
            in_specs=[
                pl.BlockSpec((BE, H), lambda i: (i, 0)),
                pl.BlockSpec((BE, H), lambda i: (i, 0)),
                pl.BlockSpec((H, H), lambda i: (0, 0)),
            ],
            out_specs=pl.BlockSpec((BE, H), lambda i: (i, 0)),
            out_shape=jax.ShapeDtypeStruct((E, H), jnp.float32),
        )(s, binput, W_h)

    a = gs_atoms(msg, aidx3d)[:N]

    woa = W_o[:, :AF]
    woh = W_o[:, AF:]
    bo2d = b_o.reshape(1, H)
    msb = jnp.broadcast_to(mol_sizes.reshape(NMOL, 1), (NMOL, H))
    out = pl.pallas_call(
        _mm_out_body,
        grid=(1,),
        in_specs=[
            pl.BlockSpec((N, AF), lambda i: (0, 0)),
            pl.BlockSpec((N, H), lambda i: (0, 0)),
            pl.BlockSpec((H, AF), lambda i: (0, 0)),
            pl.BlockSpec((H, H), lambda i: (0, 0)),
            pl.BlockSpec((1, H), lambda i: (0, 0)),
            pl.BlockSpec((NMOL, H), lambda i: (0, 0)),
        ],
        out_specs=pl.BlockSpec((NMOL, H), lambda i: (0, 0)),
        out_shape=jax.ShapeDtypeStruct((NMOL, H), jnp.float32),
    )(fatoms, a, woa, woh, bo2d, msb)
    return out
